# Initial kernel scaffold; baseline (speedup 1.0000x reference)
import jax
import jax.numpy as jnp
from jax.experimental import pallas as pl


def kernel(n0, n1, n2, user_feat_idx, item_feat_idx, user_feat_emb, item_feat_emb, user_proj_w, user_proj_b, item_proj_w, item_proj_b, w0_w, w0_b, w1_w, w1_b):
    raise NotImplementedError("write your pallas kernel here")


# speedup vs baseline: 9.1942x; 9.1942x over previous
import functools

import jax
import jax.numpy as jnp
from jax import lax
from jax.experimental import pallas as pl
from jax.experimental.pallas import tpu as pltpu
from jax.experimental.pallas import tpu_sc as plsc

B = 4096
K = 10
D = 64
NU = 100000
NI = 100000
F = 10
UFEAT = 3207
IFEAT = 2094

NC = 2
NS = 16
NW = NC * NS
NU_PAD = 100352
N_PER_W = NU_PAD // NW
CHUNK = 64
NCHUNK = N_PER_W // CHUNK

_mesh = plsc.VectorSubcoreMesh(core_axis_name="c", subcore_axis_name="s")
_sc_params = pltpu.CompilerParams(use_tc_tiling_on_sc=False)


def _wid():
  return lax.axis_index("s") * NC + lax.axis_index("c")


def _bag_body(pu, pi, uidx, iidx, bagu, bagi, idx_v, rows_v, out_v, sem):
  wid = _wid()

  def run(tbl, fidx, outp):
    def chunk(c, carry):
      i0 = (wid * N_PER_W + c * CHUNK) * F
      pltpu.sync_copy(fidx.at[pl.ds(i0, CHUNK * F)], idx_v)
      cps = [
          pltpu.async_copy(tbl.at[idx_v.at[pl.ds(g * 128, 128)]],
                           rows_v.at[pl.ds(g * 128, 128)], sem)
          for g in range(5)
      ]
      for cp in cps:
        cp.wait()

      def node(u, carry2):
        base = u * F
        for l in range(D // 16):
          sl = pl.ds(l * 16, 16)
          acc = rows_v[base, sl]
          for f in range(1, F):
            acc = acc + rows_v[base + f, sl]
          out_v[u, sl] = acc
        return carry2

      lax.fori_loop(0, CHUNK, node, 0)
      row0 = wid * N_PER_W + c * CHUNK
      pltpu.sync_copy(out_v, outp.at[pl.ds(row0, CHUNK)])
      return carry

    lax.fori_loop(0, NCHUNK, chunk, 0)

  run(pu, uidx, bagu)
  run(pi, iidx, bagi)


_bag_call = pl.kernel(
    _bag_body,
    out_type=(jax.ShapeDtypeStruct((NU_PAD, D), jnp.float32),
              jax.ShapeDtypeStruct((NU_PAD, D), jnp.float32)),
    mesh=_mesh,
    compiler_params=_sc_params,
    scratch_types=[
        pltpu.VMEM((CHUNK * F,), jnp.int32),
        pltpu.VMEM((CHUNK * F, D), jnp.float32),
        pltpu.VMEM((CHUNK, D), jnp.float32),
        pltpu.SemaphoreType.DMA,
    ],
)


def _agg_body(bagu, bagi, n0r, n1r, n2r, h0s, g1s, t2s,
              idx_v, rows_v, out_v, sem):
  wid = _wid()

  def t2chunk(c, carry):
    i0 = (wid * 128 + c * 4) * 100
    pltpu.sync_copy(n2r.at[pl.ds(i0, 400)], idx_v)
    cps = [
        pltpu.async_copy(bagu.at[idx_v.at[pl.ds(g * 80, 80)]],
                         rows_v.at[pl.ds(g * 80, 80)], sem)
        for g in range(5)
    ]
    for cp in cps:
      cp.wait()

    def tgt(t, carry2):
      base = t * 100
      for l in range(D // 16):
        sl = pl.ds(l * 16, 16)
        acc = rows_v[base, sl]
        for j in range(1, 100):
          acc = acc + rows_v[base + j, sl]
        out_v[t, sl] = acc
      return carry2

    lax.fori_loop(0, 4, tgt, 0)
    pltpu.sync_copy(out_v.at[pl.ds(0, 4)],
                    t2s.at[pl.ds(wid * 128 + c * 4, 4)])
    return carry

  lax.fori_loop(0, 32, t2chunk, 0)

  def g1chunk(c, carry):
    i0 = (wid * 128 + c * 8) * K
    pltpu.sync_copy(n1r.at[pl.ds(i0, 80)], idx_v.at[pl.ds(0, 80)])
    pltpu.async_copy(bagi.at[idx_v.at[pl.ds(0, 80)]],
                     rows_v.at[pl.ds(0, 80)], sem).wait()

    def tgt(t, carry2):
      base = t * K
      for l in range(D // 16):
        sl = pl.ds(l * 16, 16)
        acc = rows_v[base, sl]
        for j in range(1, K):
          acc = acc + rows_v[base + j, sl]
        out_v[t, sl] = acc
      return carry2

    lax.fori_loop(0, 8, tgt, 0)
    pltpu.sync_copy(out_v.at[pl.ds(0, 8)],
                    g1s.at[pl.ds(wid * 128 + c * 8, 8)])
    return carry

  lax.fori_loop(0, 16, g1chunk, 0)

  pltpu.sync_copy(n0r.at[pl.ds(wid * 128, 128)], idx_v.at[pl.ds(0, 128)])
  pltpu.async_copy(bagu.at[idx_v.at[pl.ds(0, 128)]],
                   rows_v.at[pl.ds(0, 128)], sem).wait()
  pltpu.sync_copy(rows_v.at[pl.ds(0, 128)], h0s.at[pl.ds(wid * 128, 128)])


_agg_call = pl.kernel(
    _agg_body,
    out_type=(jax.ShapeDtypeStruct((B, D), jnp.float32),
              jax.ShapeDtypeStruct((B, D), jnp.float32),
              jax.ShapeDtypeStruct((B, D), jnp.float32)),
    mesh=_mesh,
    compiler_params=_sc_params,
    scratch_types=[
        pltpu.VMEM((400,), jnp.int32),
        pltpu.VMEM((400, D), jnp.float32),
        pltpu.VMEM((8, D), jnp.float32),
        pltpu.SemaphoreType.DMA,
    ],
)


def _dg(a, b):
  return lax.dot_general(a, b, (((1,), (1,)), ((), ())),
                         preferred_element_type=jnp.float32)


def _proj_body(e_ref, w_ref, o_ref):
  o_ref[...] = _dg(e_ref[...], w_ref[...]) * (1.0 / F)


def _proj(e, w):
  rows = e.shape[0]
  pad = (-rows) % 8
  e = jnp.pad(e, ((0, pad), (0, 0)))
  return pl.pallas_call(
      _proj_body,
      out_shape=jax.ShapeDtypeStruct((rows + pad, D), jnp.float32),
  )(e, w)


def _final_body(h0_ref, g1_ref, t2_ref, w0_ref, w1_ref,
                bu_ref, bi_ref, b0_ref, b1_ref, o_ref):
  h0 = h0_ref[...] + bu_ref[...]
  g1 = g1_ref[...] + float(K) * bi_ref[...]
  t2 = t2_ref[...] + float(K * K) * bu_ref[...]
  w0 = w0_ref[...]
  w1 = w1_ref[...]
  w0a, w0b = w0[:, :D], w0[:, D:]
  w1a, w1b = w1[:, :D], w1[:, D:]
  y0 = _dg(h0, w0a) + _dg(g1, w0b) + b0_ref[...]
  z = _dg(g1, w0a) + _dg(t2, w0b) + float(K) * b0_ref[...]
  o_ref[...] = _dg(y0, w1a) + _dg(z, w1b) + b1_ref[...]


_final = pl.pallas_call(
    _final_body,
    out_shape=jax.ShapeDtypeStruct((B, D), jnp.float32),
)


@jax.jit
def kernel(n0, n1, n2, user_feat_idx, item_feat_idx, user_feat_emb,
           item_feat_emb, user_proj_w, user_proj_b, item_proj_w, item_proj_b,
           w0_w, w0_b, w1_w, w1_b):
  pu = _proj(user_feat_emb, user_proj_w)
  pi = _proj(item_feat_emb, item_proj_w)
  uidx = jnp.pad(user_feat_idx, (0, (NU_PAD - NU) * F))
  iidx = jnp.pad(item_feat_idx, (0, (NU_PAD - NI) * F))
  bagu, bagi = _bag_call(pu, pi, uidx, iidx)
  h0s, g1s, t2s = _agg_call(bagu, bagi, n0, n1, n2)
  return _final(h0s, g1s, t2s, w0_w, w1_w,
                user_proj_b.reshape(1, D), item_proj_b.reshape(1, D),
                w0_b.reshape(1, D), w1_b.reshape(1, D))


# speedup vs baseline: 11.9467x; 1.2994x over previous
import jax
import jax.numpy as jnp
from jax import lax
from jax.experimental import pallas as pl
from jax.experimental.pallas import tpu as pltpu
from jax.experimental.pallas import tpu_sc as plsc

B = 4096
K = 10
D = 64
NU = 100000
NI = 100000
F = 10
UFEAT = 3207
IFEAT = 2094

NC = 2
NS = 16
NW = NC * NS
NU_PAD = 100352
N_PER_W = NU_PAD // NW
CHUNK = 56
NCHUNK = N_PER_W // CHUNK

_mesh = plsc.VectorSubcoreMesh(core_axis_name="c", subcore_axis_name="s")
_sc_params = pltpu.CompilerParams(use_tc_tiling_on_sc=False)


def _wid():
  return lax.axis_index("s") * NC + lax.axis_index("c")


def _fire(tbl, idx_v, rows_v, sem, nrows):
  for g in range(nrows // 80):
    pltpu.async_copy(tbl.at[idx_v.at[pl.ds(g * 80, 80)]],
                     rows_v.at[pl.ds(g * 80, 80)], sem)


def _drain(tbl, idx_v, rows_v, sem, nrows):
  for g in range(nrows // 80):
    pltpu.make_async_copy(tbl.at[idx_v.at[pl.ds(g * 80, 80)]],
                          rows_v.at[pl.ds(g * 80, 80)], sem).wait()


def _reduce(rows_v, out_v, nodes, r):
  def node(u, carry):
    base = u * r
    for l in range(D // 16):
      sl = pl.ds(l * 16, 16)
      acc = rows_v[base, sl]
      for j in range(1, r):
        acc = acc + rows_v[base + j, sl]
      out_v[u, sl] = acc
    return carry

  lax.fori_loop(0, nodes, node, 0)


def _gather_sum_pipeline(tbl, fidx, outp, bufs, *, nchunks, nodes, r,
                         idx0_fn, orow_fn):
  idx_b, rows_b, out_b, sem_b = bufs
  nrows = nodes * r

  def fetch(c, p):
    pltpu.sync_copy(fidx.at[pl.ds(idx0_fn(c), nrows)],
                    idx_b[p].at[pl.ds(0, nrows)])
    _fire(tbl, idx_b[p], rows_b[p], sem_b[p], nrows)

  def consume(c, p):
    _drain(tbl, idx_b[p], rows_b[p], sem_b[p], nrows)
    _reduce(rows_b[p], out_b[p], nodes, r)
    pltpu.sync_copy(out_b[p].at[pl.ds(0, nodes)],
                    outp.at[pl.ds(orow_fn(c), nodes)])

  fetch(0, 0)

  def pair(i, carry):
    c0 = 2 * i
    fetch(c0 + 1, 1)
    consume(c0, 0)

    @pl.when(c0 + 2 < nchunks)
    def _():
      fetch(c0 + 2, 0)

    consume(c0 + 1, 1)
    return carry

  lax.fori_loop(0, nchunks // 2, pair, 0)


def _bag_body(pu, pi, uidx, iidx, bagu, bagi,
              idx_a, idx_bb, rows_a, rows_bb, out_a, out_bb, sem_a, sem_bb):
  wid = _wid()
  bufs = ((idx_a, idx_bb), (rows_a, rows_bb), (out_a, out_bb), (sem_a, sem_bb))

  def run(tbl, fidx, outp):
    _gather_sum_pipeline(
        tbl, fidx, outp, bufs, nchunks=NCHUNK, nodes=CHUNK, r=F,
        idx0_fn=lambda c: (wid * N_PER_W + c * CHUNK) * F,
        orow_fn=lambda c: wid * N_PER_W + c * CHUNK)

  run(pu, uidx, bagu)
  run(pi, iidx, bagi)


_bag_call = pl.kernel(
    _bag_body,
    out_type=(jax.ShapeDtypeStruct((NU_PAD, D), jnp.float32),
              jax.ShapeDtypeStruct((NU_PAD, D), jnp.float32)),
    mesh=_mesh,
    compiler_params=_sc_params,
    scratch_types=[
        pltpu.VMEM((CHUNK * F,), jnp.int32),
        pltpu.VMEM((CHUNK * F,), jnp.int32),
        pltpu.VMEM((CHUNK * F, D), jnp.float32),
        pltpu.VMEM((CHUNK * F, D), jnp.float32),
        pltpu.VMEM((CHUNK, D), jnp.float32),
        pltpu.VMEM((CHUNK, D), jnp.float32),
        pltpu.SemaphoreType.DMA,
        pltpu.SemaphoreType.DMA,
    ],
)


def _agg_body(bagu, bagi, n0r, n1r, n2r, h0s, g1s, t2s,
              idx_a, idx_bb, rows_a, rows_bb, out_a, out_bb, sem_a, sem_bb):
  wid = _wid()
  bufs = ((idx_a, idx_bb), (rows_a, rows_bb), (out_a, out_bb), (sem_a, sem_bb))

  _gather_sum_pipeline(
      bagu, n2r, t2s, bufs, nchunks=32, nodes=4, r=100,
      idx0_fn=lambda c: (wid * 128 + c * 4) * 100,
      orow_fn=lambda c: wid * 128 + c * 4)

  _gather_sum_pipeline(
      bagi, n1r, g1s, bufs, nchunks=16, nodes=8, r=K,
      idx0_fn=lambda c: (wid * 128 + c * 8) * K,
      orow_fn=lambda c: wid * 128 + c * 8)

  pltpu.sync_copy(n0r.at[pl.ds(wid * 128, 128)], idx_a.at[pl.ds(0, 128)])
  for g in range(2):
    pltpu.async_copy(bagu.at[idx_a.at[pl.ds(g * 64, 64)]],
                     rows_a.at[pl.ds(g * 64, 64)], sem_a)
  for g in range(2):
    pltpu.make_async_copy(bagu.at[idx_a.at[pl.ds(g * 64, 64)]],
                          rows_a.at[pl.ds(g * 64, 64)], sem_a).wait()
  pltpu.sync_copy(rows_a.at[pl.ds(0, 128)], h0s.at[pl.ds(wid * 128, 128)])


_agg_call = pl.kernel(
    _agg_body,
    out_type=(jax.ShapeDtypeStruct((B, D), jnp.float32),
              jax.ShapeDtypeStruct((B, D), jnp.float32),
              jax.ShapeDtypeStruct((B, D), jnp.float32)),
    mesh=_mesh,
    compiler_params=_sc_params,
    scratch_types=[
        pltpu.VMEM((400,), jnp.int32),
        pltpu.VMEM((400,), jnp.int32),
        pltpu.VMEM((400, D), jnp.float32),
        pltpu.VMEM((400, D), jnp.float32),
        pltpu.VMEM((8, D), jnp.float32),
        pltpu.VMEM((8, D), jnp.float32),
        pltpu.SemaphoreType.DMA,
        pltpu.SemaphoreType.DMA,
    ],
)


def _dg(a, b):
  return lax.dot_general(a, b, (((1,), (1,)), ((), ())),
                         preferred_element_type=jnp.float32)


def _proj_body(e_ref, w_ref, o_ref):
  o_ref[...] = _dg(e_ref[...], w_ref[...]) * (1.0 / F)


def _proj(e, w):
  rows = e.shape[0]
  pad = (-rows) % 8
  e = jnp.pad(e, ((0, pad), (0, 0)))
  return pl.pallas_call(
      _proj_body,
      out_shape=jax.ShapeDtypeStruct((rows + pad, D), jnp.float32),
  )(e, w)


def _final_body(h0_ref, g1_ref, t2_ref, w0_ref, w1_ref,
                bu_ref, bi_ref, b0_ref, b1_ref, o_ref):
  h0 = h0_ref[...] + bu_ref[...]
  g1 = g1_ref[...] + float(K) * bi_ref[...]
  t2 = t2_ref[...] + float(K * K) * bu_ref[...]
  w0 = w0_ref[...]
  w1 = w1_ref[...]
  w0a, w0b = w0[:, :D], w0[:, D:]
  w1a, w1b = w1[:, :D], w1[:, D:]
  y0 = _dg(h0, w0a) + _dg(g1, w0b) + b0_ref[...]
  z = _dg(g1, w0a) + _dg(t2, w0b) + float(K) * b0_ref[...]
  o_ref[...] = _dg(y0, w1a) + _dg(z, w1b) + b1_ref[...]


_final = pl.pallas_call(
    _final_body,
    out_shape=jax.ShapeDtypeStruct((B, D), jnp.float32),
)


@jax.jit
def kernel(n0, n1, n2, user_feat_idx, item_feat_idx, user_feat_emb,
           item_feat_emb, user_proj_w, user_proj_b, item_proj_w, item_proj_b,
           w0_w, w0_b, w1_w, w1_b):
  pu = _proj(user_feat_emb, user_proj_w)
  pi = _proj(item_feat_emb, item_proj_w)
  uidx = jnp.pad(user_feat_idx, (0, (NU_PAD - NU) * F))
  iidx = jnp.pad(item_feat_idx, (0, (NU_PAD - NI) * F))
  bagu, bagi = _bag_call(pu, pi, uidx, iidx)
  h0s, g1s, t2s = _agg_call(bagu, bagi, n0, n1, n2)
  return _final(h0s, g1s, t2s, w0_w, w1_w,
                user_proj_b.reshape(1, D), item_proj_b.reshape(1, D),
                w0_b.reshape(1, D), w1_b.reshape(1, D))


# speedup vs baseline: 20.7345x; 1.7356x over previous
import jax
import jax.numpy as jnp
from jax import lax
from jax.experimental import pallas as pl
from jax.experimental.pallas import tpu as pltpu
from jax.experimental.pallas import tpu_sc as plsc

B = 4096
K = 10
D = 64
NU = 100000
NI = 100000
F = 10
UFEAT = 3207
IFEAT = 2094

NC = 2
NS = 16
NW = NC * NS
NU_PAD = 100352
N_PER_W = NU_PAD // NW
CHUNK = 56
NCHUNK = N_PER_W // CHUNK

_mesh = plsc.VectorSubcoreMesh(core_axis_name="c", subcore_axis_name="s")
_sc_params = pltpu.CompilerParams(use_tc_tiling_on_sc=False, needs_layout_passes=False)


def _wid():
  return lax.axis_index("s") * NC + lax.axis_index("c")


def _fire(tbl, idx_v, rows_v, sem, nrows):
  for g in range(nrows // 80):
    pltpu.async_copy(tbl.at[idx_v.at[pl.ds(g * 80, 80)]],
                     rows_v.at[pl.ds(g * 80, 80)], sem)


def _drain(tbl, idx_v, rows_v, sem, nrows):
  for g in range(nrows // 80):
    pltpu.make_async_copy(tbl.at[idx_v.at[pl.ds(g * 80, 80)]],
                          rows_v.at[pl.ds(g * 80, 80)], sem).wait()


def _reduce(rows_v, out_v, nodes, r):
  def node(u, carry):
    base = u * r
    for h in range(D // 32):
      sl = pl.ds(h * 32, 32)
      acc_a, acc_b = plsc.unpack(rows_v[base, sl],
                                 format=plsc.PackFormat.INTERLEAVED)
      for j in range(1, r):
        aj, bj = plsc.unpack(rows_v[base + j, sl],
                             format=plsc.PackFormat.INTERLEAVED)
        acc_a = acc_a + aj
        acc_b = acc_b + bj
      out_v[u, sl] = plsc.pack(acc_a, acc_b,
                               format=plsc.PackFormat.INTERLEAVED)
    return carry

  lax.fori_loop(0, nodes, node, 0)


def _gather_sum_pipeline(tbl, fidx, outp, bufs, *, nchunks, nodes, r,
                         idx0_fn, orow_fn):
  idx_b, rows_b, out_b, sem_b = bufs
  nrows = nodes * r

  def fetch(c, p):
    pltpu.sync_copy(fidx.at[pl.ds(idx0_fn(c), nrows)],
                    idx_b[p].at[pl.ds(0, nrows)])
    _fire(tbl, idx_b[p], rows_b[p], sem_b[p], nrows)

  def consume(c, p):
    _drain(tbl, idx_b[p], rows_b[p], sem_b[p], nrows)
    _reduce(rows_b[p], out_b[p], nodes, r)
    pltpu.sync_copy(out_b[p].at[pl.ds(0, nodes)],
                    outp.at[pl.ds(orow_fn(c), nodes)])

  fetch(0, 0)

  def pair(i, carry):
    c0 = 2 * i
    fetch(c0 + 1, 1)
    consume(c0, 0)

    @pl.when(c0 + 2 < nchunks)
    def _():
      fetch(c0 + 2, 0)

    consume(c0 + 1, 1)
    return carry

  lax.fori_loop(0, nchunks // 2, pair, 0)


def _bag_body(pu, pi, uidx, iidx, bagu, bagi,
              idx_a, idx_bb, rows_a, rows_bb, out_a, out_bb, sem_a, sem_bb):
  wid = _wid()
  bufs = ((idx_a, idx_bb), (rows_a, rows_bb), (out_a, out_bb), (sem_a, sem_bb))

  def run(tbl, fidx, outp):
    _gather_sum_pipeline(
        tbl, fidx, outp, bufs, nchunks=NCHUNK, nodes=CHUNK, r=F,
        idx0_fn=lambda c: (wid * N_PER_W + c * CHUNK) * F,
        orow_fn=lambda c: wid * N_PER_W + c * CHUNK)

  run(pu, uidx, bagu)
  run(pi, iidx, bagi)


_bag_call = pl.kernel(
    _bag_body,
    out_type=(jax.ShapeDtypeStruct((NU_PAD, D), jnp.bfloat16),
              jax.ShapeDtypeStruct((NU_PAD, D), jnp.bfloat16)),
    mesh=_mesh,
    compiler_params=_sc_params,
    scratch_types=[
        pltpu.VMEM((CHUNK * F,), jnp.int32),
        pltpu.VMEM((CHUNK * F,), jnp.int32),
        pltpu.VMEM((CHUNK * F, D), jnp.bfloat16),
        pltpu.VMEM((CHUNK * F, D), jnp.bfloat16),
        pltpu.VMEM((CHUNK, D), jnp.bfloat16),
        pltpu.VMEM((CHUNK, D), jnp.bfloat16),
        pltpu.SemaphoreType.DMA,
        pltpu.SemaphoreType.DMA,
    ],
)


def _agg_body(bagu, bagi, n0r, n1r, n2r, h0s, g1s, t2s,
              idx_a, idx_bb, rows_a, rows_bb, out_a, out_bb, sem_a, sem_bb):
  wid = _wid()
  bufs = ((idx_a, idx_bb), (rows_a, rows_bb), (out_a, out_bb), (sem_a, sem_bb))

  _gather_sum_pipeline(
      bagu, n2r, t2s, bufs, nchunks=32, nodes=4, r=100,
      idx0_fn=lambda c: (wid * 128 + c * 4) * 100,
      orow_fn=lambda c: wid * 128 + c * 4)

  _gather_sum_pipeline(
      bagi, n1r, g1s, bufs, nchunks=16, nodes=8, r=K,
      idx0_fn=lambda c: (wid * 128 + c * 8) * K,
      orow_fn=lambda c: wid * 128 + c * 8)

  pltpu.sync_copy(n0r.at[pl.ds(wid * 128, 128)], idx_a.at[pl.ds(0, 128)])
  for g in range(2):
    pltpu.async_copy(bagu.at[idx_a.at[pl.ds(g * 64, 64)]],
                     rows_a.at[pl.ds(g * 64, 64)], sem_a)
  for g in range(2):
    pltpu.make_async_copy(bagu.at[idx_a.at[pl.ds(g * 64, 64)]],
                          rows_a.at[pl.ds(g * 64, 64)], sem_a).wait()
  pltpu.sync_copy(rows_a.at[pl.ds(0, 128)], h0s.at[pl.ds(wid * 128, 128)])


_agg_call = pl.kernel(
    _agg_body,
    out_type=(jax.ShapeDtypeStruct((B, D), jnp.bfloat16),
              jax.ShapeDtypeStruct((B, D), jnp.bfloat16),
              jax.ShapeDtypeStruct((B, D), jnp.bfloat16)),
    mesh=_mesh,
    compiler_params=_sc_params,
    scratch_types=[
        pltpu.VMEM((400,), jnp.int32),
        pltpu.VMEM((400,), jnp.int32),
        pltpu.VMEM((400, D), jnp.bfloat16),
        pltpu.VMEM((400, D), jnp.bfloat16),
        pltpu.VMEM((8, D), jnp.bfloat16),
        pltpu.VMEM((8, D), jnp.bfloat16),
        pltpu.SemaphoreType.DMA,
        pltpu.SemaphoreType.DMA,
    ],
)


def _dg(a, b):
  return lax.dot_general(a, b, (((1,), (1,)), ((), ())),
                         preferred_element_type=jnp.float32)


def _proj_body(e_ref, w_ref, o_ref):
  o_ref[...] = (_dg(e_ref[...], w_ref[...]) * (1.0 / F)).astype(jnp.bfloat16)


def _proj(e, w):
  rows = e.shape[0]
  pad = (-rows) % 8
  e = jnp.pad(e, ((0, pad), (0, 0)))
  return pl.pallas_call(
      _proj_body,
      out_shape=jax.ShapeDtypeStruct((rows + pad, D), jnp.bfloat16),
  )(e, w)


def _final_body(h0_ref, g1_ref, t2_ref, w0_ref, w1_ref,
                bu_ref, bi_ref, b0_ref, b1_ref, o_ref):
  h0 = h0_ref[...].astype(jnp.float32) + bu_ref[...]
  g1 = g1_ref[...].astype(jnp.float32) + float(K) * bi_ref[...]
  t2 = t2_ref[...].astype(jnp.float32) + float(K * K) * bu_ref[...]
  w0 = w0_ref[...]
  w1 = w1_ref[...]
  w0a, w0b = w0[:, :D], w0[:, D:]
  w1a, w1b = w1[:, :D], w1[:, D:]
  y0 = _dg(h0, w0a) + _dg(g1, w0b) + b0_ref[...]
  z = _dg(g1, w0a) + _dg(t2, w0b) + float(K) * b0_ref[...]
  o_ref[...] = _dg(y0, w1a) + _dg(z, w1b) + b1_ref[...]


_final = pl.pallas_call(
    _final_body,
    out_shape=jax.ShapeDtypeStruct((B, D), jnp.float32),
)


@jax.jit
def kernel(n0, n1, n2, user_feat_idx, item_feat_idx, user_feat_emb,
           item_feat_emb, user_proj_w, user_proj_b, item_proj_w, item_proj_b,
           w0_w, w0_b, w1_w, w1_b):
  pu = _proj(user_feat_emb, user_proj_w)
  pi = _proj(item_feat_emb, item_proj_w)
  uidx = jnp.pad(user_feat_idx, (0, (NU_PAD - NU) * F))
  iidx = jnp.pad(item_feat_idx, (0, (NU_PAD - NI) * F))
  bagu, bagi = _bag_call(pu, pi, uidx, iidx)
  h0s, g1s, t2s = _agg_call(bagu, bagi, n0, n1, n2)
  return _final(h0s, g1s, t2s, w0_w, w1_w,
                user_proj_b.reshape(1, D), item_proj_b.reshape(1, D),
                w0_b.reshape(1, D), w1_b.reshape(1, D))


# speedup vs baseline: 21.4497x; 1.0345x over previous
import jax
import jax.numpy as jnp
from jax import lax
from jax.experimental import pallas as pl
from jax.experimental.pallas import tpu as pltpu
from jax.experimental.pallas import tpu_sc as plsc

B = 4096
K = 10
D = 64
NU = 100000
NI = 100000
F = 10
UFEAT = 3207
IFEAT = 2094

NC = 2
NS = 16
NW = NC * NS
NU_PAD = 100352
N_PER_W = NU_PAD // NW
CHUNK = 56
NCHUNK = N_PER_W // CHUNK
UFEAT_PAD = 3208
IFEAT_PAD = 2096

_mesh = plsc.VectorSubcoreMesh(core_axis_name="c", subcore_axis_name="s")
_sc_params = pltpu.CompilerParams(use_tc_tiling_on_sc=False, needs_layout_passes=False)


def _wid():
  return lax.axis_index("s") * NC + lax.axis_index("c")


def _fire(tbl, idx_v, rows_v, sem, nrows):
  for g in range(nrows // 80):
    pltpu.async_copy(tbl.at[idx_v.at[pl.ds(g * 80, 80)]],
                     rows_v.at[pl.ds(g * 80, 80)], sem)


def _drain(tbl, idx_v, rows_v, sem, nrows):
  for g in range(nrows // 80):
    pltpu.make_async_copy(tbl.at[idx_v.at[pl.ds(g * 80, 80)]],
                          rows_v.at[pl.ds(g * 80, 80)], sem).wait()


def _tree(vals):
  while len(vals) > 1:
    nxt = [vals[i] + vals[i + 1] for i in range(0, len(vals) - 1, 2)]
    if len(vals) % 2:
      nxt.append(vals[-1])
    vals = nxt
  return vals[0]


def _reduce(rows_v, out_v, nodes, r):
  def node(u, carry):
    base = u * r
    for h in range(D // 32):
      sl = pl.ds(h * 32, 32)
      acc_a = None
      acc_b = None
      for j0 in range(0, r, 8):
        terms = [plsc.unpack(rows_v[base + j, sl],
                             format=plsc.PackFormat.INTERLEAVED)
                 for j in range(j0, min(j0 + 8, r))]
        ta = _tree([t[0] for t in terms])
        tb = _tree([t[1] for t in terms])
        acc_a = ta if acc_a is None else acc_a + ta
        acc_b = tb if acc_b is None else acc_b + tb
      out_v[u, sl] = plsc.pack(acc_a, acc_b,
                               format=plsc.PackFormat.INTERLEAVED)
    return carry

  lax.fori_loop(0, nodes, node, 0)


def _gather_sum_pipeline(tbl, fidx, outp, bufs, *, nchunks, nodes, r,
                         idx0_fn, orow_fn):
  idx_b, rows_b, out_b, sem_b = bufs
  nrows = nodes * r

  def fetch(c, p):
    pltpu.sync_copy(fidx.at[pl.ds(idx0_fn(c), nrows)],
                    idx_b[p].at[pl.ds(0, nrows)])
    _fire(tbl, idx_b[p], rows_b[p], sem_b[p], nrows)

  def consume(c, p):
    _drain(tbl, idx_b[p], rows_b[p], sem_b[p], nrows)
    _reduce(rows_b[p], out_b[p], nodes, r)
    pltpu.sync_copy(out_b[p].at[pl.ds(0, nodes)],
                    outp.at[pl.ds(orow_fn(c), nodes)])

  fetch(0, 0)

  def pair(i, carry):
    c0 = 2 * i
    fetch(c0 + 1, 1)
    consume(c0, 0)

    @pl.when(c0 + 2 < nchunks)
    def _():
      fetch(c0 + 2, 0)

    consume(c0 + 1, 1)
    return carry

  lax.fori_loop(0, nchunks // 2, pair, 0)


def _bag_body(pu, pi, uidx, iidx, bagu, bagi,
              idx_a, idx_bb, rows_a, rows_bb, out_a, out_bb, sem_a, sem_bb):
  wid = _wid()
  bufs = ((idx_a, idx_bb), (rows_a, rows_bb), (out_a, out_bb), (sem_a, sem_bb))

  def run(tbl, fidx, outp):
    _gather_sum_pipeline(
        tbl, fidx, outp, bufs, nchunks=NCHUNK, nodes=CHUNK, r=F,
        idx0_fn=lambda c: (wid * N_PER_W + c * CHUNK) * F,
        orow_fn=lambda c: wid * N_PER_W + c * CHUNK)

  run(pu, uidx, bagu)
  run(pi, iidx, bagi)


_bag_call = pl.kernel(
    _bag_body,
    out_type=(jax.ShapeDtypeStruct((NU_PAD, D), jnp.bfloat16),
              jax.ShapeDtypeStruct((NU_PAD, D), jnp.bfloat16)),
    mesh=_mesh,
    compiler_params=_sc_params,
    scratch_types=[
        pltpu.VMEM((CHUNK * F,), jnp.int32),
        pltpu.VMEM((CHUNK * F,), jnp.int32),
        pltpu.VMEM((CHUNK * F, D), jnp.bfloat16),
        pltpu.VMEM((CHUNK * F, D), jnp.bfloat16),
        pltpu.VMEM((CHUNK, D), jnp.bfloat16),
        pltpu.VMEM((CHUNK, D), jnp.bfloat16),
        pltpu.SemaphoreType.DMA,
        pltpu.SemaphoreType.DMA,
    ],
)


def _agg_body(bagu, bagi, n0r, n1r, n2r, h0s, g1s, t2s,
              idx_a, idx_bb, rows_a, rows_bb, out_a, out_bb, sem_a, sem_bb):
  wid = _wid()
  bufs = ((idx_a, idx_bb), (rows_a, rows_bb), (out_a, out_bb), (sem_a, sem_bb))

  _gather_sum_pipeline(
      bagu, n2r, t2s, bufs, nchunks=32, nodes=4, r=100,
      idx0_fn=lambda c: (wid * 128 + c * 4) * 100,
      orow_fn=lambda c: wid * 128 + c * 4)

  _gather_sum_pipeline(
      bagi, n1r, g1s, bufs, nchunks=16, nodes=8, r=K,
      idx0_fn=lambda c: (wid * 128 + c * 8) * K,
      orow_fn=lambda c: wid * 128 + c * 8)

  pltpu.sync_copy(n0r.at[pl.ds(wid * 128, 128)], idx_a.at[pl.ds(0, 128)])
  for g in range(2):
    pltpu.async_copy(bagu.at[idx_a.at[pl.ds(g * 64, 64)]],
                     rows_a.at[pl.ds(g * 64, 64)], sem_a)
  for g in range(2):
    pltpu.make_async_copy(bagu.at[idx_a.at[pl.ds(g * 64, 64)]],
                          rows_a.at[pl.ds(g * 64, 64)], sem_a).wait()
  pltpu.sync_copy(rows_a.at[pl.ds(0, 128)], h0s.at[pl.ds(wid * 128, 128)])


_agg_call = pl.kernel(
    _agg_body,
    out_type=(jax.ShapeDtypeStruct((B, D), jnp.bfloat16),
              jax.ShapeDtypeStruct((B, D), jnp.bfloat16),
              jax.ShapeDtypeStruct((B, D), jnp.bfloat16)),
    mesh=_mesh,
    compiler_params=_sc_params,
    scratch_types=[
        pltpu.VMEM((400,), jnp.int32),
        pltpu.VMEM((400,), jnp.int32),
        pltpu.VMEM((400, D), jnp.bfloat16),
        pltpu.VMEM((400, D), jnp.bfloat16),
        pltpu.VMEM((8, D), jnp.bfloat16),
        pltpu.VMEM((8, D), jnp.bfloat16),
        pltpu.SemaphoreType.DMA,
        pltpu.SemaphoreType.DMA,
    ],
)


def _dg(a, b):
  return lax.dot_general(a, b, (((1,), (1,)), ((), ())),
                         preferred_element_type=jnp.float32)


def _proj_body(e_ref, w_ref, o_ref):
  o_ref[...] = (_dg(e_ref[...], w_ref[...]) * (1.0 / F)).astype(jnp.bfloat16)


def _proj(e, w):
  rows = e.shape[0]
  pad = (-rows) % 8
  e = jnp.pad(e, ((0, pad), (0, 0)))
  return pl.pallas_call(
      _proj_body,
      out_shape=jax.ShapeDtypeStruct((rows + pad, D), jnp.bfloat16),
  )(e, w)


def _final_body(h0_ref, g1_ref, t2_ref, w0_ref, w1_ref,
                bu_ref, bi_ref, b0_ref, b1_ref, o_ref):
  h0 = h0_ref[...].astype(jnp.float32) + bu_ref[...]
  g1 = g1_ref[...].astype(jnp.float32) + float(K) * bi_ref[...]
  t2 = t2_ref[...].astype(jnp.float32) + float(K * K) * bu_ref[...]
  w0 = w0_ref[...]
  w1 = w1_ref[...]
  w0a, w0b = w0[:, :D], w0[:, D:]
  w1a, w1b = w1[:, :D], w1[:, D:]
  y0 = _dg(h0, w0a) + _dg(g1, w0b) + b0_ref[...]
  z = _dg(g1, w0a) + _dg(t2, w0b) + float(K) * b0_ref[...]
  o_ref[...] = _dg(y0, w1a) + _dg(z, w1b) + b1_ref[...]


_final = pl.pallas_call(
    _final_body,
    out_shape=jax.ShapeDtypeStruct((B, D), jnp.float32),
)


@jax.jit
def kernel(n0, n1, n2, user_feat_idx, item_feat_idx, user_feat_emb,
           item_feat_emb, user_proj_w, user_proj_b, item_proj_w, item_proj_b,
           w0_w, w0_b, w1_w, w1_b):
  pu = _proj(user_feat_emb, user_proj_w)
  pi = _proj(item_feat_emb, item_proj_w)
  uidx = jnp.pad(user_feat_idx, (0, (NU_PAD - NU) * F))
  iidx = jnp.pad(item_feat_idx, (0, (NU_PAD - NI) * F))
  bagu, bagi = _bag_call(pu, pi, uidx, iidx)
  h0s, g1s, t2s = _agg_call(bagu, bagi, n0, n1, n2)
  return _final(h0s, g1s, t2s, w0_w, w1_w,
                user_proj_b.reshape(1, D), item_proj_b.reshape(1, D),
                w0_b.reshape(1, D), w1_b.reshape(1, D))
